# baseline (speedup 1.0000x reference)
import jax
import jax.numpy as jnp
import numpy as np
from jax.experimental import pallas as pl

P = 512
EMB = 32
H = 64
N = P * P
N2 = N // 2
R = 2048


def _cell_kernel(corr_ref, mask_ref, ht_ref, ct_ref, wemb_ref, bemb_ref,
                 sel_ref, wih_ref, whh_ref, b_ref, hout_ref, cout_ref):
    ht = ht_ref[...]
    ct = ct_ref[...]
    emb = jnp.maximum(
        jnp.dot(corr_ref[...], wemb_ref[...],
                preferred_element_type=jnp.float32) + bemb_ref[...],
        0.0)
    m = jnp.dot(mask_ref[...], sel_ref[...],
                preferred_element_type=jnp.float32)
    gates = (jnp.dot(emb, wih_ref[...], preferred_element_type=jnp.float32) +
             jnp.dot(ht, whh_ref[...], preferred_element_type=jnp.float32) +
             b_ref[...])
    i_g = jax.nn.sigmoid(gates[:, 0:128])
    f_g = jax.nn.sigmoid(gates[:, 128:256])
    g_g = jnp.tanh(gates[:, 256:384])
    o_g = jax.nn.sigmoid(gates[:, 384:512])
    c_new = f_g * ct + i_g * g_g
    h_new = o_g * jnp.tanh(c_new)
    hout_ref[...] = ht + m * (h_new - ht)
    cout_ref[...] = ct + m * (c_new - ct)


def _pack_weights(W_emb, b_emb, W_ih, b_ih, W_hh, b_hh):
    zemb = jnp.zeros((2, EMB), dtype=jnp.float32)
    wembT = W_emb.T
    wemb2 = jnp.concatenate([
        jnp.concatenate([wembT, zemb], axis=1),
        jnp.concatenate([zemb, wembT], axis=1),
    ], axis=0)
    bemb2 = jnp.concatenate([b_emb, b_emb]).reshape(1, 2 * EMB)

    def dup(w):
        k = w.shape[0]
        z = jnp.zeros_like(w)
        top = jnp.concatenate([w, z], axis=1).reshape(k, 2, 4, H)
        bot = jnp.concatenate([z, w], axis=1).reshape(k, 2, 4, H)
        both = jnp.concatenate([top, bot], axis=0)
        return both.transpose(0, 2, 1, 3).reshape(2 * k, 8 * H)

    wih2 = dup(W_ih.T)
    whh2 = dup(W_hh.T)
    b = (b_ih + b_hh).reshape(4, H)
    b2 = jnp.concatenate([b, b], axis=1).reshape(1, 8 * H)
    sel = jnp.concatenate([
        jnp.concatenate([jnp.ones((1, H)), jnp.zeros((1, H))], axis=1),
        jnp.concatenate([jnp.zeros((1, H)), jnp.ones((1, H))], axis=1),
    ], axis=0).astype(jnp.float32)
    return wemb2, bemb2, sel, wih2, whh2, b2


def kernel(corr_index, rela_ht, rela_ct, nei_index, W_emb, b_emb, W_ih, b_ih,
           W_hh, b_hh):
    corr = corr_index.reshape(N2, 4)
    ht = rela_ht.reshape(N2, 2 * H)
    ct = rela_ct.reshape(N2, 2 * H)
    mask = nei_index.reshape(N2, 2).astype(jnp.float32)
    wemb2, bemb2, sel, wih2, whh2, b2 = _pack_weights(
        W_emb, b_emb, W_ih, b_ih, W_hh, b_hh)

    ht_out, ct_out = pl.pallas_call(
        _cell_kernel,
        grid=(N2 // R,),
        in_specs=[
            pl.BlockSpec((R, 4), lambda i: (i, 0)),
            pl.BlockSpec((R, 2), lambda i: (i, 0)),
            pl.BlockSpec((R, 2 * H), lambda i: (i, 0)),
            pl.BlockSpec((R, 2 * H), lambda i: (i, 0)),
            pl.BlockSpec((4, 2 * EMB), lambda i: (0, 0)),
            pl.BlockSpec((1, 2 * EMB), lambda i: (0, 0)),
            pl.BlockSpec((2, 2 * H), lambda i: (0, 0)),
            pl.BlockSpec((2 * EMB, 8 * H), lambda i: (0, 0)),
            pl.BlockSpec((2 * H, 8 * H), lambda i: (0, 0)),
            pl.BlockSpec((1, 8 * H), lambda i: (0, 0)),
        ],
        out_specs=[
            pl.BlockSpec((R, 2 * H), lambda i: (i, 0)),
            pl.BlockSpec((R, 2 * H), lambda i: (i, 0)),
        ],
        out_shape=[jax.ShapeDtypeStruct((N2, 2 * H), jnp.float32)] * 2,
    )(corr, mask, ht, ct, wemb2, bemb2, sel, wih2, whh2, b2)
    return ht_out.reshape(P, P, H), ct_out.reshape(P, P, H)


# speedup vs baseline: 1.6941x; 1.6941x over previous
import jax
import jax.numpy as jnp
from jax.experimental import pallas as pl

P = 512
EMB = 32
H = 64
N = P * P
R = 4096


def _cell_kernel(aux_ref, ht_ref, ct_ref, wfront_ref, bfront_ref,
                 wih_ref, whh_ref, b_ref, hout_ref, cout_ref):
    ht = ht_ref[...]
    ct = ct_ref[...]
    front = jnp.maximum(
        jnp.dot(aux_ref[...], wfront_ref[...],
                preferred_element_type=jnp.float32) + bfront_ref[...],
        0.0)
    emb = front[:, :EMB]
    m = front[:, EMB:]
    gates = (jnp.dot(emb, wih_ref[...], preferred_element_type=jnp.float32) +
             jnp.dot(ht, whh_ref[...], preferred_element_type=jnp.float32) +
             b_ref[...])
    i_g = jax.nn.sigmoid(gates[:, 0 * H:1 * H])
    f_g = jax.nn.sigmoid(gates[:, 1 * H:2 * H])
    g_g = jnp.tanh(gates[:, 2 * H:3 * H])
    o_g = jax.nn.sigmoid(gates[:, 3 * H:4 * H])
    c_new = f_g * ct + i_g * g_g
    h_new = o_g * jnp.tanh(c_new)
    hout_ref[...] = ht + m * (h_new - ht)
    cout_ref[...] = ct + m * (c_new - ct)


def kernel(corr_index, rela_ht, rela_ct, nei_index, W_emb, b_emb, W_ih, b_ih,
           W_hh, b_hh):
    ht = rela_ht.reshape(N, H)
    ct = rela_ct.reshape(N, H)
    aux = jnp.concatenate([
        corr_index.reshape(N, 2),
        nei_index.reshape(N, 1).astype(jnp.float32),
        jnp.zeros((N, 1), dtype=jnp.float32),
    ], axis=1)
    wfront = jnp.zeros((4, EMB + H), dtype=jnp.float32)
    wfront = wfront.at[0:2, :EMB].set(W_emb.T)
    wfront = wfront.at[2, EMB:].set(1.0)
    bfront = jnp.concatenate([b_emb, jnp.zeros((H,), jnp.float32)])
    bfront = bfront.reshape(1, EMB + H)
    b = (b_ih + b_hh).reshape(1, 4 * H)

    ht_out, ct_out = pl.pallas_call(
        _cell_kernel,
        grid=(N // R,),
        in_specs=[
            pl.BlockSpec((R, 4), lambda i: (i, 0)),
            pl.BlockSpec((R, H), lambda i: (i, 0)),
            pl.BlockSpec((R, H), lambda i: (i, 0)),
            pl.BlockSpec((4, EMB + H), lambda i: (0, 0)),
            pl.BlockSpec((1, EMB + H), lambda i: (0, 0)),
            pl.BlockSpec((EMB, 4 * H), lambda i: (0, 0)),
            pl.BlockSpec((H, 4 * H), lambda i: (0, 0)),
            pl.BlockSpec((1, 4 * H), lambda i: (0, 0)),
        ],
        out_specs=[
            pl.BlockSpec((R, H), lambda i: (i, 0)),
            pl.BlockSpec((R, H), lambda i: (i, 0)),
        ],
        out_shape=[jax.ShapeDtypeStruct((N, H), jnp.float32)] * 2,
    )(aux, ht, ct, wfront, bfront, W_ih.T, W_hh.T, b)
    return ht_out.reshape(P, P, H), ct_out.reshape(P, P, H)


# speedup vs baseline: 5.3658x; 3.1674x over previous
import functools
import jax
import jax.numpy as jnp
from jax.experimental import pallas as pl

P = 512
EMB = 32
H = 64
B = 8


def _cell_kernel(aux_ref, ht_ref, ct_ref, wfront_ref, bfront_ref,
                 wih_ref, whh_ref, b_ref, hout_ref, cout_ref):
    wfront = wfront_ref[...]
    bfront = bfront_ref[...]
    wih = wih_ref[...]
    whh = whh_ref[...]
    b = b_ref[...]
    for k in range(B):
        ht = ht_ref[k]
        ct = ct_ref[k]
        front = jnp.maximum(
            jnp.dot(wfront, aux_ref[k],
                    preferred_element_type=jnp.float32) + bfront,
            0.0)
        emb = front[:EMB, :]
        m = front[EMB:, :]
        gates = (jnp.dot(wih, emb, preferred_element_type=jnp.float32) +
                 jnp.dot(whh, ht, preferred_element_type=jnp.float32) + b)
        i_g = jax.nn.sigmoid(gates[0 * H:1 * H, :])
        f_g = jax.nn.sigmoid(gates[1 * H:2 * H, :])
        g_g = jnp.tanh(gates[2 * H:3 * H, :])
        o_g = jax.nn.sigmoid(gates[3 * H:4 * H, :])
        c_new = f_g * ct + i_g * g_g
        h_new = o_g * jnp.tanh(c_new)
        hout_ref[k] = ht + m * (h_new - ht)
        cout_ref[k] = ct + m * (c_new - ct)


def kernel(corr_index, rela_ht, rela_ct, nei_index, W_emb, b_emb, W_ih, b_ih,
           W_hh, b_hh):
    htT = rela_ht.transpose(0, 2, 1)
    ctT = rela_ct.transpose(0, 2, 1)
    aux = jnp.concatenate([
        corr_index.transpose(0, 2, 1),
        nei_index.astype(jnp.float32)[:, None, :],
    ], axis=1)
    wfront = jnp.zeros((EMB + H, 3), dtype=jnp.float32)
    wfront = wfront.at[:EMB, 0:2].set(W_emb)
    wfront = wfront.at[EMB:, 2].set(1.0)
    bfront = jnp.concatenate([b_emb, jnp.zeros((H,), jnp.float32)])
    bfront = bfront.reshape(EMB + H, 1)
    b = (b_ih + b_hh).reshape(4 * H, 1)

    ht_out, ct_out = pl.pallas_call(
        _cell_kernel,
        grid=(P // B,),
        in_specs=[
            pl.BlockSpec((B, 3, P), lambda i: (i, 0, 0)),
            pl.BlockSpec((B, H, P), lambda i: (i, 0, 0)),
            pl.BlockSpec((B, H, P), lambda i: (i, 0, 0)),
            pl.BlockSpec((EMB + H, 3), lambda i: (0, 0)),
            pl.BlockSpec((EMB + H, 1), lambda i: (0, 0)),
            pl.BlockSpec((4 * H, EMB), lambda i: (0, 0)),
            pl.BlockSpec((4 * H, H), lambda i: (0, 0)),
            pl.BlockSpec((4 * H, 1), lambda i: (0, 0)),
        ],
        out_specs=[
            pl.BlockSpec((B, H, P), lambda i: (i, 0, 0)),
            pl.BlockSpec((B, H, P), lambda i: (i, 0, 0)),
        ],
        out_shape=[jax.ShapeDtypeStruct((P, H, P), jnp.float32)] * 2,
    )(aux, htT, ctT, wfront, bfront, W_ih, W_hh, b)
    return ht_out.transpose(0, 2, 1), ct_out.transpose(0, 2, 1)


# speedup vs baseline: 5.7021x; 1.0627x over previous
import jax
import jax.numpy as jnp
from jax.experimental import pallas as pl

P = 512
EMB = 32
H = 64
B = 16


def _cell_kernel(aux_ref, ht_ref, ct_ref, wfront_ref, bfront_ref,
                 wih_ref, whh_ref, b_ref, hout_ref, cout_ref):
    wfront = wfront_ref[...]
    bfront = bfront_ref[...]
    wih = wih_ref[...]
    whh = whh_ref[...]
    b = b_ref[...]
    for k in range(B):
        ht = ht_ref[k]
        ct = ct_ref[k]
        front = jnp.maximum(
            jnp.dot(wfront, aux_ref[k],
                    preferred_element_type=jnp.float32) + bfront,
            0.0)
        emb = front[:EMB, :]
        m = front[EMB:, :]
        gates = (jnp.dot(wih, emb, preferred_element_type=jnp.float32) +
                 jnp.dot(whh, ht, preferred_element_type=jnp.float32) + b)
        t = jnp.tanh(gates)
        sig = 0.5 + 0.5 * t[0:3 * H, :]
        i_g = sig[0 * H:1 * H, :]
        f_g = sig[1 * H:2 * H, :]
        o_g = sig[2 * H:3 * H, :]
        g_g = t[3 * H:4 * H, :]
        c_new = f_g * ct + i_g * g_g
        h_new = o_g * jnp.tanh(c_new)
        hout_ref[k] = ht + m * (h_new - ht)
        cout_ref[k] = ct + m * (c_new - ct)


def _prep_gate_weights(W_ih, b_ih, W_hh, b_hh):
    def reorder(w):
        g4 = w.reshape(4, H, -1)
        return jnp.concatenate(
            [0.5 * g4[0], 0.5 * g4[1], 0.5 * g4[3], g4[2]], axis=0)

    wih = reorder(W_ih)
    whh = reorder(W_hh)
    b = reorder((b_ih + b_hh)[:, None])
    return wih, whh, b


def kernel(corr_index, rela_ht, rela_ct, nei_index, W_emb, b_emb, W_ih, b_ih,
           W_hh, b_hh):
    htT = rela_ht.transpose(0, 2, 1)
    ctT = rela_ct.transpose(0, 2, 1)
    aux = jnp.concatenate([
        corr_index.transpose(0, 2, 1),
        nei_index.astype(jnp.float32)[:, None, :],
    ], axis=1)
    wfront = jnp.zeros((EMB + H, 3), dtype=jnp.float32)
    wfront = wfront.at[:EMB, 0:2].set(W_emb)
    wfront = wfront.at[EMB:, 2].set(1.0)
    bfront = jnp.concatenate([b_emb, jnp.zeros((H,), jnp.float32)])
    bfront = bfront.reshape(EMB + H, 1)
    wih, whh, b = _prep_gate_weights(W_ih, b_ih, W_hh, b_hh)

    ht_out, ct_out = pl.pallas_call(
        _cell_kernel,
        grid=(P // B,),
        in_specs=[
            pl.BlockSpec((B, 3, P), lambda i: (i, 0, 0)),
            pl.BlockSpec((B, H, P), lambda i: (i, 0, 0)),
            pl.BlockSpec((B, H, P), lambda i: (i, 0, 0)),
            pl.BlockSpec((EMB + H, 3), lambda i: (0, 0)),
            pl.BlockSpec((EMB + H, 1), lambda i: (0, 0)),
            pl.BlockSpec((4 * H, EMB), lambda i: (0, 0)),
            pl.BlockSpec((4 * H, H), lambda i: (0, 0)),
            pl.BlockSpec((4 * H, 1), lambda i: (0, 0)),
        ],
        out_specs=[
            pl.BlockSpec((B, H, P), lambda i: (i, 0, 0)),
            pl.BlockSpec((B, H, P), lambda i: (i, 0, 0)),
        ],
        out_shape=[jax.ShapeDtypeStruct((P, H, P), jnp.float32)] * 2,
    )(aux, htT, ctT, wfront, bfront, wih, whh, b)
    return ht_out.transpose(0, 2, 1), ct_out.transpose(0, 2, 1)
